# initial kernel scaffold (unmeasured)
import jax
import jax.numpy as jnp
from jax import lax
from jax.experimental import pallas as pl
from jax.experimental.pallas import tpu as pltpu


def kernel(
    x,
):
    def body(*refs):
        pass

    out_shape = jax.ShapeDtypeStruct(..., jnp.float32)
    return pl.pallas_call(body, out_shape=out_shape)(...)


# baseline (device time: 23538 ns/iter reference)
import jax
import jax.numpy as jnp
from jax import lax
from jax.experimental import pallas as pl
from jax.experimental.pallas import tpu as pltpu

N_DEV = 8


def kernel(x):
    m, n = x.shape

    def body(x_ref, o_ref, halo_ref, send_sems, recv_sems):
        my = lax.axis_index("i")
        left = lax.rem(my + (N_DEV - 1), N_DEV)
        right = lax.rem(my + 1, N_DEV)

        to_right = pltpu.make_async_remote_copy(
            src_ref=x_ref.at[pl.ds(m - 1, 1), :],
            dst_ref=halo_ref.at[pl.ds(0, 1), :],
            send_sem=send_sems.at[0],
            recv_sem=recv_sems.at[0],
            device_id=(right,),
            device_id_type=pl.DeviceIdType.MESH,
        )
        to_left = pltpu.make_async_remote_copy(
            src_ref=x_ref.at[pl.ds(0, 1), :],
            dst_ref=halo_ref.at[pl.ds(1, 1), :],
            send_sem=send_sems.at[1],
            recv_sem=recv_sems.at[1],
            device_id=(left,),
            device_id_type=pl.DeviceIdType.MESH,
        )
        to_right.start()
        to_left.start()

        o_ref[pl.ds(1, m - 2), :] = (
            0.25 * x_ref[pl.ds(0, m - 2), :]
            + 0.5 * x_ref[pl.ds(1, m - 2), :]
            + 0.25 * x_ref[pl.ds(2, m - 2), :]
        )

        to_right.wait()
        to_left.wait()

        @pl.when(my == 0)
        def _():
            o_ref[pl.ds(0, 1), :] = x_ref[pl.ds(0, 1), :]

        @pl.when(my != 0)
        def _():
            o_ref[pl.ds(0, 1), :] = (
                0.25 * halo_ref[pl.ds(0, 1), :]
                + 0.5 * x_ref[pl.ds(0, 1), :]
                + 0.25 * x_ref[pl.ds(1, 1), :]
            )

        @pl.when(my == N_DEV - 1)
        def _():
            o_ref[pl.ds(m - 1, 1), :] = x_ref[pl.ds(m - 1, 1), :]

        @pl.when(my != N_DEV - 1)
        def _():
            o_ref[pl.ds(m - 1, 1), :] = (
                0.25 * x_ref[pl.ds(m - 2, 1), :]
                + 0.5 * x_ref[pl.ds(m - 1, 1), :]
                + 0.25 * halo_ref[pl.ds(1, 1), :]
            )

    return pl.pallas_call(
        body,
        out_shape=jax.ShapeDtypeStruct((m, n), x.dtype),
        in_specs=[pl.BlockSpec(memory_space=pltpu.VMEM)],
        out_specs=pl.BlockSpec(memory_space=pltpu.VMEM),
        scratch_shapes=[
            pltpu.VMEM((2, n), x.dtype),
            pltpu.SemaphoreType.DMA((2,)),
            pltpu.SemaphoreType.DMA((2,)),
        ],
    )(x)


# device time: 18419 ns/iter; 1.2779x vs baseline; 1.2779x over previous
import jax
import jax.numpy as jnp
from jax import lax
from jax.experimental import pallas as pl
from jax.experimental.pallas import tpu as pltpu

N_DEV = 8
NB = 4


def kernel(x):
    m, n = x.shape
    bs = m // NB
    order = [1, 0] + list(range(2, NB))

    def body(
        x_ref,
        o_ref,
        ibuf,
        obuf,
        halo_ref,
        edge_out,
        in_sems,
        out_sems,
        send_sems,
        recv_sems,
        edge_sems,
    ):
        my = lax.axis_index("i")
        left = lax.rem(my + (N_DEV - 1), N_DEV)
        right = lax.rem(my + 1, N_DEV)

        to_right = pltpu.make_async_remote_copy(
            src_ref=x_ref.at[pl.ds(m - 8, 8), :],
            dst_ref=halo_ref.at[0],
            send_sem=send_sems.at[0],
            recv_sem=recv_sems.at[0],
            device_id=(right,),
            device_id_type=pl.DeviceIdType.MESH,
        )
        to_left = pltpu.make_async_remote_copy(
            src_ref=x_ref.at[pl.ds(0, 8), :],
            dst_ref=halo_ref.at[1],
            send_sem=send_sems.at[1],
            recv_sem=recv_sems.at[1],
            device_id=(left,),
            device_id_type=pl.DeviceIdType.MESH,
        )

        def in_desc(b, slot):
            start = b * bs
            lo = max(start - 8, 0)
            hi = min(start + bs + 8, m)
            return pltpu.make_async_copy(
                x_ref.at[pl.ds(lo, hi - lo), :],
                ibuf.at[slot, pl.ds(lo - (start - 8), hi - lo), :],
                in_sems.at[slot],
            )

        def out_desc(b, slot):
            if b == 0:
                return pltpu.make_async_copy(
                    obuf.at[slot, pl.ds(8, bs - 8), :],
                    o_ref.at[pl.ds(8, bs - 8), :],
                    out_sems.at[slot],
                )
            if b == NB - 1:
                return pltpu.make_async_copy(
                    obuf.at[slot, pl.ds(0, bs - 8), :],
                    o_ref.at[pl.ds(b * bs, bs - 8), :],
                    out_sems.at[slot],
                )
            return pltpu.make_async_copy(
                obuf.at[slot],
                o_ref.at[pl.ds(b * bs, bs), :],
                out_sems.at[slot],
            )

        in_desc(order[0], 0).start()
        in_desc(order[1], 1).start()

        barrier_sem = pltpu.get_barrier_semaphore()
        for nbr in (left, right):
            pl.semaphore_signal(
                barrier_sem,
                inc=1,
                device_id=(nbr,),
                device_id_type=pl.DeviceIdType.MESH,
            )
        pl.semaphore_wait(barrier_sem, 2)
        to_right.start()
        to_left.start()

        for idx, b in enumerate(order):
            slot = idx % 2
            if 2 <= idx + 1 < NB:
                in_desc(order[idx + 1], (idx + 1) % 2).start()
            in_desc(b, slot).wait()
            if idx >= 2:
                out_desc(order[idx - 2], slot).wait()

            obuf[slot, :, :] = (
                0.25 * ibuf[slot, pl.ds(7, bs), :]
                + 0.5 * ibuf[slot, pl.ds(8, bs), :]
                + 0.25 * ibuf[slot, pl.ds(9, bs), :]
            )

            if b == 0:
                edge_out[0, :, :] = obuf[slot, pl.ds(0, 8), :]
                edge_out[2, pl.ds(0, 2), :] = ibuf[slot, pl.ds(8, 2), :]
            if b == NB - 1:
                edge_out[1, :, :] = obuf[slot, pl.ds(bs - 8, 8), :]
                edge_out[2, pl.ds(2, 2), :] = ibuf[slot, pl.ds(bs + 6, 2), :]

            out_desc(b, slot).start()

        out_desc(order[NB - 2], (NB - 2) % 2).wait()
        out_desc(order[NB - 1], (NB - 1) % 2).wait()

        to_right.wait_recv()

        @pl.when(my == 0)
        def _():
            edge_out[0, pl.ds(0, 1), :] = edge_out[2, pl.ds(0, 1), :]

        @pl.when(my != 0)
        def _():
            edge_out[0, pl.ds(0, 1), :] = (
                0.25 * halo_ref[0, pl.ds(7, 1), :]
                + 0.5 * edge_out[2, pl.ds(0, 1), :]
                + 0.25 * edge_out[2, pl.ds(1, 1), :]
            )

        to_left.wait_recv()

        @pl.when(my == N_DEV - 1)
        def _():
            edge_out[1, pl.ds(7, 1), :] = edge_out[2, pl.ds(3, 1), :]

        @pl.when(my != N_DEV - 1)
        def _():
            edge_out[1, pl.ds(7, 1), :] = (
                0.25 * edge_out[2, pl.ds(2, 1), :]
                + 0.5 * edge_out[2, pl.ds(3, 1), :]
                + 0.25 * halo_ref[1, pl.ds(0, 1), :]
            )

        top = pltpu.make_async_copy(
            edge_out.at[0], o_ref.at[pl.ds(0, 8), :], edge_sems.at[0]
        )
        bot = pltpu.make_async_copy(
            edge_out.at[1], o_ref.at[pl.ds(m - 8, 8), :], edge_sems.at[1]
        )
        top.start()
        bot.start()
        top.wait()
        bot.wait()

        to_right.wait_send()
        to_left.wait_send()

    return pl.pallas_call(
        body,
        out_shape=jax.ShapeDtypeStruct((m, n), x.dtype),
        in_specs=[pl.BlockSpec(memory_space=pl.ANY)],
        out_specs=pl.BlockSpec(memory_space=pl.ANY),
        scratch_shapes=[
            pltpu.VMEM((2, bs + 16, n), x.dtype),
            pltpu.VMEM((2, bs, n), x.dtype),
            pltpu.VMEM((2, 8, n), x.dtype),
            pltpu.VMEM((3, 8, n), x.dtype),
            pltpu.SemaphoreType.DMA((2,)),
            pltpu.SemaphoreType.DMA((2,)),
            pltpu.SemaphoreType.DMA((2,)),
            pltpu.SemaphoreType.DMA((2,)),
            pltpu.SemaphoreType.DMA((2,)),
        ],
        compiler_params=pltpu.CompilerParams(collective_id=0),
    )(x)


# device time: 16938 ns/iter; 1.3897x vs baseline; 1.0874x over previous
import jax
import jax.numpy as jnp
from jax import lax
from jax.experimental import pallas as pl
from jax.experimental.pallas import tpu as pltpu

N_DEV = 8
NB = 4


def kernel(x):
    m, n = x.shape
    bs = m // NB
    order = [1, 0] + list(range(2, NB))

    def body(
        x_ref,
        o_ref,
        ibuf,
        obuf,
        halo_ref,
        edge_out,
        in_sems,
        out_sems,
        send_sems,
        recv_sems,
        edge_sems,
    ):
        my = lax.axis_index("i")
        left = lax.rem(my + (N_DEV - 1), N_DEV)
        right = lax.rem(my + 1, N_DEV)

        to_right = pltpu.make_async_remote_copy(
            src_ref=x_ref.at[pl.ds(m - 8, 8), :],
            dst_ref=halo_ref.at[0],
            send_sem=send_sems.at[0],
            recv_sem=recv_sems.at[0],
            device_id=(right,),
            device_id_type=pl.DeviceIdType.MESH,
        )
        to_left = pltpu.make_async_remote_copy(
            src_ref=x_ref.at[pl.ds(0, 8), :],
            dst_ref=halo_ref.at[1],
            send_sem=send_sems.at[1],
            recv_sem=recv_sems.at[1],
            device_id=(left,),
            device_id_type=pl.DeviceIdType.MESH,
        )

        def in_desc(b, slot):
            start = b * bs
            lo = max(start - 8, 0)
            hi = min(start + bs + 8, m)
            return pltpu.make_async_copy(
                x_ref.at[pl.ds(lo, hi - lo), :],
                ibuf.at[slot, pl.ds(lo - (start - 8), hi - lo), :],
                in_sems.at[slot],
            )

        def out_desc(b, slot):
            if b == 0:
                return pltpu.make_async_copy(
                    obuf.at[slot, pl.ds(8, bs - 8), :],
                    o_ref.at[pl.ds(8, bs - 8), :],
                    out_sems.at[slot],
                )
            if b == NB - 1:
                return pltpu.make_async_copy(
                    obuf.at[slot, pl.ds(0, bs - 8), :],
                    o_ref.at[pl.ds(b * bs, bs - 8), :],
                    out_sems.at[slot],
                )
            return pltpu.make_async_copy(
                obuf.at[slot],
                o_ref.at[pl.ds(b * bs, bs), :],
                out_sems.at[slot],
            )

        in_desc(order[0], 0).start()
        in_desc(order[1], 1).start()

        barrier_sem = pltpu.get_barrier_semaphore()
        for nbr in (left, right):
            pl.semaphore_signal(
                barrier_sem,
                inc=1,
                device_id=(nbr,),
                device_id_type=pl.DeviceIdType.MESH,
            )
        pl.semaphore_wait(barrier_sem, 2)
        to_right.start()
        to_left.start()

        for idx, b in enumerate(order):
            slot = idx % 2
            if 2 <= idx + 1 < NB:
                in_desc(order[idx + 1], (idx + 1) % 2).start()
            in_desc(b, slot).wait()
            if idx >= 2:
                out_desc(order[idx - 2], slot).wait()

            obuf[slot, :, :] = (
                0.25 * ibuf[slot, pl.ds(7, bs), :]
                + 0.5 * ibuf[slot, pl.ds(8, bs), :]
                + 0.25 * ibuf[slot, pl.ds(9, bs), :]
            )

            if b == 0:
                edge_out[0, :, :] = obuf[slot, pl.ds(0, 8), :]
                edge_out[2, pl.ds(0, 2), :] = ibuf[slot, pl.ds(8, 2), :]
            if b == NB - 1:
                edge_out[1, :, :] = obuf[slot, pl.ds(bs - 8, 8), :]
                edge_out[2, pl.ds(2, 2), :] = ibuf[slot, pl.ds(bs + 6, 2), :]

            out_desc(b, slot).start()

        to_right.wait_recv()

        @pl.when(my == 0)
        def _():
            edge_out[0, pl.ds(0, 1), :] = edge_out[2, pl.ds(0, 1), :]

        @pl.when(my != 0)
        def _():
            edge_out[0, pl.ds(0, 1), :] = (
                0.25 * halo_ref[0, pl.ds(7, 1), :]
                + 0.5 * edge_out[2, pl.ds(0, 1), :]
                + 0.25 * edge_out[2, pl.ds(1, 1), :]
            )

        to_left.wait_recv()

        @pl.when(my == N_DEV - 1)
        def _():
            edge_out[1, pl.ds(7, 1), :] = edge_out[2, pl.ds(3, 1), :]

        @pl.when(my != N_DEV - 1)
        def _():
            edge_out[1, pl.ds(7, 1), :] = (
                0.25 * edge_out[2, pl.ds(2, 1), :]
                + 0.5 * edge_out[2, pl.ds(3, 1), :]
                + 0.25 * halo_ref[1, pl.ds(0, 1), :]
            )

        top = pltpu.make_async_copy(
            edge_out.at[0], o_ref.at[pl.ds(0, 8), :], edge_sems.at[0]
        )
        bot = pltpu.make_async_copy(
            edge_out.at[1], o_ref.at[pl.ds(m - 8, 8), :], edge_sems.at[1]
        )
        top.start()
        bot.start()

        out_desc(order[NB - 2], (NB - 2) % 2).wait()
        out_desc(order[NB - 1], (NB - 1) % 2).wait()
        top.wait()
        bot.wait()
        to_right.wait_send()
        to_left.wait_send()

    return pl.pallas_call(
        body,
        out_shape=jax.ShapeDtypeStruct((m, n), x.dtype),
        in_specs=[pl.BlockSpec(memory_space=pl.ANY)],
        out_specs=pl.BlockSpec(memory_space=pl.ANY),
        scratch_shapes=[
            pltpu.VMEM((2, bs + 16, n), x.dtype),
            pltpu.VMEM((2, bs, n), x.dtype),
            pltpu.VMEM((2, 8, n), x.dtype),
            pltpu.VMEM((3, 8, n), x.dtype),
            pltpu.SemaphoreType.DMA((2,)),
            pltpu.SemaphoreType.DMA((2,)),
            pltpu.SemaphoreType.DMA((2,)),
            pltpu.SemaphoreType.DMA((2,)),
            pltpu.SemaphoreType.DMA((2,)),
        ],
        compiler_params=pltpu.CompilerParams(collective_id=0),
    )(x)
